# baseline (speedup 1.0000x reference)
import jax
import jax.numpy as jnp
from jax.experimental import pallas as pl
from jax.experimental.pallas import tpu as pltpu

_BM = 512
_NBUF = 4


def _make_body(bm, d_model):
    def body(x0, x1, x2, x3, mask_ref, w_ref, probs_ref, logits_ref,
             b0, b1, b2, b3, acc_ref, s0, s1, s2, s3):
        i = pl.program_id(0)
        nsteps = pl.num_programs(0)
        srcs = (x0, x1, x2, x3)
        bufs = (b0, b1, b2, b3)
        sems = (s0, s1, s2, s3)

        def _copy(step, slot):
            return pltpu.make_async_copy(
                srcs[slot].at[pl.ds(step * bm, bm), :],
                bufs[slot],
                sems[slot],
            )

        @pl.when(i == 0)
        def _prologue():
            for s in range(_NBUF - 1):
                _copy(s, s).start()

        nxt = i + _NBUF - 1
        for s in range(_NBUF):
            @pl.when(jnp.logical_and(nxt < nsteps,
                                     jax.lax.rem(nxt, _NBUF) == s))
            def _refill(s=s):
                _copy(nxt, s).start()

        for s in range(_NBUF):
            @pl.when(jax.lax.rem(i, _NBUF) == s)
            def _consume(s=s):
                _copy(i, s).wait()
                acc_ref[...] = jax.lax.dot_general(
                    bufs[s][...], w_ref[...], (((1,), (0,)), ((), ())),
                    preferred_element_type=jnp.float32)

        logits = acc_ref[...]
        m = jnp.max(logits, axis=-1, keepdims=True)
        e = jnp.exp(logits - m)
        p = e / jnp.sum(e, axis=-1, keepdims=True)
        probs_ref[...] = p * mask_ref[...]
        logits_ref[...] = logits

    return body


def kernel(inputs, padding_mask, num_experts, w):
    del num_experts
    inputs = inputs.astype(jnp.float32)
    tokens, d_model = inputs.shape
    n_experts = w.shape[1]
    bm = _BM
    probs, logits = pl.pallas_call(
        _make_body(bm, d_model),
        grid=(tokens // bm,),
        in_specs=[pl.BlockSpec(memory_space=pl.ANY)] * 4 + [
            pl.BlockSpec((bm, 1), lambda i: (i, 0)),
            pl.BlockSpec((d_model, n_experts), lambda i: (0, 0)),
        ],
        out_specs=[
            pl.BlockSpec((bm, n_experts), lambda i: (i, 0)),
            pl.BlockSpec((bm, n_experts), lambda i: (i, 0)),
        ],
        out_shape=[
            jax.ShapeDtypeStruct((tokens, n_experts), jnp.float32),
            jax.ShapeDtypeStruct((tokens, n_experts), jnp.float32),
        ],
        scratch_shapes=(
            [pltpu.VMEM((bm, d_model), jnp.float32)] * 4
            + [pltpu.VMEM((bm, 64), jnp.float32)]
            + [pltpu.SemaphoreType.DMA] * 4
        ),
        compiler_params=pltpu.CompilerParams(
            dimension_semantics=("arbitrary",),
        ),
    )(inputs, inputs, inputs, inputs,
      padding_mask.astype(jnp.float32), w.astype(jnp.float32))
    return (probs, logits)
